# baseline (speedup 1.0000x reference)
import functools

import jax
import jax.numpy as jnp
from jax import lax
from jax.experimental import pallas as pl
from jax.experimental.pallas import tpu as pltpu
from jax.experimental.pallas import tpu_sc as plsc

VOCAB = 1000000
H = 64
S = 200
B = 4096
T = 4
L = 16
NC = 2
NS = 16
NW = NC * NS
BPW = B // NW
NCHUNK = BPW // L
NSTRIPE = VOCAB // 128
TAIL = VOCAB - NSTRIPE * 128
NPAIR = VOCAB // 2


def _iota(off=0):
    return lax.iota(jnp.int32, L) + off


SUP = 3
SUPW = SUP * 128
NSUP = NSTRIPE // SUP
SUP_BASE = NSUP // NW
SUP_EXTRA = NSUP % NW


def _tr_body(tableT_hbm, p_hbm, in0, in1, out0, out1,
             isem0, isem1, osem0, osem1):
    wid = lax.axis_index("s") * NC + lax.axis_index("c")
    ins, outs = [in0, in1], [out0, out1]
    isems, osems = [isem0, isem1], [osem0, osem1]

    s0 = SUP_BASE * wid + jnp.minimum(wid, SUP_EXTRA)
    cnt = SUP_BASE + jnp.where(wid < SUP_EXTRA, 1, 0)

    rowv = [lax.shift_right_logical(_iota(cc * L), 1) for cc in range(8)]
    parv = [(_iota(cc * L) & 1) * H for cc in range(8)]

    def start_in(k, b):
        pltpu.async_copy(tableT_hbm.at[:, pl.ds((s0 + k) * SUPW, SUPW)],
                         ins[b], isems[b])

    def wait_in(b):
        pltpu.make_async_copy(tableT_hbm.at[:, pl.ds(0, SUPW)],
                              ins[b], isems[b]).wait()

    def wait_out(b):
        pltpu.make_async_copy(p_hbm.at[pl.ds(0, SUPW // 2), :],
                              outs[b], osems[b]).wait()

    start_in(0, 0)
    start_in(1, 1)

    def step(s2, carry):
        for b in range(2):
            k = 2 * s2 + b

            @pl.when(k < cnt)
            def _(k=k, b=b):
                wait_in(b)

                @pl.when(k >= 2)
                def _():
                    wait_out(b)

                for ss in range(SUP):
                    osl = outs[b].at[pl.ds(ss * 64, 64)]

                    def hb(h, c, b=b, ss=ss, osl=osl):
                        for cc in range(8):
                            vals = ins[b][h, pl.ds(ss * 128 + cc * L, L)]
                            plsc.store_scatter(osl, [rowv[cc], parv[cc] + h],
                                               vals)
                        return c
                    lax.fori_loop(0, H, hb, 0, unroll=4)

                pltpu.async_copy(
                    outs[b],
                    p_hbm.at[pl.ds((s0 + k) * (SUPW // 2), SUPW // 2), :],
                    osems[b])

                @pl.when(k + 2 < cnt)
                def _():
                    start_in(k + 2, b)
        return carry

    lax.fori_loop(0, (SUP_BASE + 2) // 2, step, 0)
    wait_out(0)
    wait_out(1)


_transpose = functools.partial(
    pl.kernel,
    out_type=jax.ShapeDtypeStruct((NPAIR, 2 * H), jnp.float32),
    mesh=plsc.VectorSubcoreMesh(core_axis_name="c", subcore_axis_name="s"),
    compiler_params=pltpu.CompilerParams(needs_layout_passes=False),
    scratch_types=[
        pltpu.VMEM((H, SUPW), jnp.float32),
        pltpu.VMEM((H, SUPW), jnp.float32),
        pltpu.VMEM((SUPW // 2, 2 * H), jnp.float32),
        pltpu.VMEM((SUPW // 2, 2 * H), jnp.float32),
        pltpu.SemaphoreType.DMA,
        pltpu.SemaphoreType.DMA,
        pltpu.SemaphoreType.DMA,
        pltpu.SemaphoreType.DMA,
    ],
)(_tr_body)


def _sc_body(seqs_hbm, tgt_hbm, pairs_hbm, tail_hbm, w_hbm, out_hbm,
             seqs_l, tgt_v, tok_v, pair_v, tail_v, tiles_v, w_v, wbc_v,
             pred_v, sem):
    wid = lax.axis_index("s") * NC + lax.axis_index("c")
    base = wid * BPW
    CUT = NSTRIPE * 128

    pltpu.sync_copy(seqs_hbm.at[:, pl.ds(base, BPW)], seqs_l)
    pltpu.sync_copy(tgt_hbm.at[:, pl.ds(base, BPW)], tgt_v)
    pltpu.sync_copy(w_hbm, w_v)
    pltpu.sync_copy(tail_hbm, tail_v)

    iidx = [_iota(c * L) for c in range(NCHUNK)]
    for t in range(T):
        for c in range(NCHUNK):
            sl = pl.ds(c * L, L)
            tok = plsc.load_gather(seqs_l, [tgt_v[t, sl], iidx[c]])
            tok_v[t, sl] = tok
            pair_v[t, sl] = jnp.where(tok >= CUT, 0,
                                      lax.shift_right_logical(tok, 1))

    def wfill(j, carry):
        wbc_v[j, :] = plsc.load_gather(w_v, [jnp.full((L,), j, jnp.int32)])
        return carry
    lax.fori_loop(0, T * H, wfill, 0)

    cps = [pltpu.async_copy(pairs_hbm.at[pair_v.at[t]],
                            tiles_v.at[pl.ds(t * BPW, BPW)], sem)
           for t in range(T)]
    for cp in cps:
        cp.wait()

    for t in range(T):
        kidx = [_iota(t * BPW + c * L) for c in range(NCHUNK)]
        toks = [tok_v[t, pl.ds(c * L, L)] for c in range(NCHUNK)]
        parcol = [(tk & 1) * H for tk in toks]
        istail = [tk >= CUT for tk in toks]
        tbase = [jnp.maximum(tk - CUT, 0) * H for tk in toks]

        def hbody(h, accs, t=t, kidx=kidx, parcol=parcol, istail=istail,
                  tbase=tbase):
            bw = wbc_v[t * H + h, :]
            return tuple(
                accs[c] + jnp.where(
                    istail[c],
                    plsc.load_gather(tail_v, [tbase[c] + h]),
                    plsc.load_gather(tiles_v, [kidx[c], parcol[c] + h]))
                * bw
                for c in range(NCHUNK))

        accs = lax.fori_loop(
            0, H, hbody, tuple(jnp.zeros((L,), jnp.float32)
                               for _ in range(NCHUNK)))
        for c in range(NCHUNK):
            sl = pl.ds(c * L, L)
            if t == 0:
                pred_v[sl] = accs[c]
            else:
                pred_v[sl] = pred_v[sl] + accs[c]

    pltpu.sync_copy(pred_v, out_hbm.at[pl.ds(base, BPW)])


_sc_gather = functools.partial(
    pl.kernel,
    out_type=jax.ShapeDtypeStruct((B,), jnp.float32),
    mesh=plsc.VectorSubcoreMesh(core_axis_name="c", subcore_axis_name="s"),
    compiler_params=pltpu.CompilerParams(needs_layout_passes=False),
    scratch_types=[
        pltpu.VMEM((S, BPW), jnp.int32),
        pltpu.VMEM((T, BPW), jnp.int32),
        pltpu.VMEM((T, BPW), jnp.int32),
        pltpu.VMEM((T, BPW), jnp.int32),
        pltpu.VMEM((TAIL * H,), jnp.float32),
        pltpu.VMEM((T * BPW, 2 * H), jnp.float32),
        pltpu.VMEM((T * H,), jnp.float32),
        pltpu.VMEM((T * H, L), jnp.float32),
        pltpu.VMEM((BPW,), jnp.float32),
        pltpu.SemaphoreType.DMA,
    ],
)(_sc_body)


def _loss_body(pred_ref, lab_ref, b_ref, out_ref):
    p = pred_ref[:] + b_ref[0]
    lab = lab_ref[:]
    terms = (jnp.maximum(p, 0.0) - p * lab
             + jnp.log(1.0 + jnp.exp(-jnp.abs(p))))
    out_ref[0, 0] = jnp.sum(terms) * (1.0 / B)


_loss_call = pl.pallas_call(
    _loss_body,
    out_shape=jax.ShapeDtypeStruct((1, 1), jnp.float32),
    in_specs=[
        pl.BlockSpec(memory_space=pltpu.VMEM),
        pl.BlockSpec(memory_space=pltpu.VMEM),
        pl.BlockSpec(memory_space=pltpu.SMEM),
    ],
    out_specs=pl.BlockSpec(memory_space=pltpu.SMEM),
)


def kernel(seqs, targets, labels, table, W, b):
    seqs32 = seqs.astype(jnp.int32)
    w_flat = W.reshape(-1)
    tail = table[NSTRIPE * 128:, :].reshape(-1)
    pairs = _transpose(table.T)
    pred = _sc_gather(seqs32, targets, pairs, tail, w_flat)
    loss = _loss_call(pred.reshape(B // 128, 128),
                      labels.reshape(B // 128, 128), b)
    return loss[0, 0]


# speedup vs baseline: 2.4981x; 2.4981x over previous
import functools

import jax
import jax.numpy as jnp
from jax import lax
from jax.experimental import pallas as pl
from jax.experimental.pallas import tpu as pltpu
from jax.experimental.pallas import tpu_sc as plsc

VOCAB = 1000000
H = 64
S = 200
B = 4096
T = 4
L = 16
NC = 2
NS = 16
NW = NC * NS
BPW = B // NW
NCHUNK = BPW // L
ROWS = T * BPW


def _sc_body(seqs_hbm, tgt_hbm, table_hbm, w_hbm, out_hbm,
             idx_v, tok_v, rows_v, w_v, wbc_v, pred_v, sem):
    wid = lax.axis_index("s") * NC + lax.axis_index("c")
    base = wid * BPW

    for t in range(T):
        pltpu.sync_copy(tgt_hbm.at[pl.ds(t * B + base, BPW)], idx_v.at[t])
    pltpu.sync_copy(w_hbm, w_v)
    for t in range(T):
        for j in range(NCHUNK):
            col = lax.iota(jnp.int32, L) + (base + j * L)
            sl = pl.ds(j * L, L)
            idx_v[t, sl] = idx_v[t, sl] * B + col

    cps = [pltpu.async_copy(seqs_hbm.at[idx_v.at[t]], tok_v.at[t], sem)
           for t in range(T)]
    for cp in cps:
        cp.wait()

    def wfill(j, carry):
        wbc_v[j, :] = plsc.load_gather(w_v, [jnp.full((L,), j, jnp.int32)])
        return carry
    lax.fori_loop(0, T * H, wfill, 0)

    def fetch(k, carry):
        t = k // NCHUNK
        tokc = tok_v[t, pl.ds((k % NCHUNK) * L, L)]
        for l in range(L):
            pltpu.async_copy(table_hbm.at[tokc[l]], rows_v.at[k * L + l], sem)
        return carry
    lax.fori_loop(0, T * NCHUNK, fetch, 0)
    pltpu.make_async_copy(table_hbm.at[pl.ds(0, ROWS)], rows_v, sem).wait()

    for t in range(T):
        iidx = [lax.iota(jnp.int32, L) + (t * BPW + c * L)
                for c in range(NCHUNK)]

        def hbody(h, accs, t=t, iidx=iidx):
            bw = wbc_v[t * H + h, :]
            colh = jnp.full((L,), h, jnp.int32)
            return tuple(
                accs[c] + plsc.load_gather(rows_v, [iidx[c], colh]) * bw
                for c in range(NCHUNK))

        accs = lax.fori_loop(
            0, H, hbody, tuple(jnp.zeros((L,), jnp.float32)
                               for _ in range(NCHUNK)))
        for c in range(NCHUNK):
            sl = pl.ds(c * L, L)
            if t == 0:
                pred_v[sl] = accs[c]
            else:
                pred_v[sl] = pred_v[sl] + accs[c]

    pltpu.sync_copy(pred_v, out_hbm.at[pl.ds(base, BPW)])


_sc_gather = functools.partial(
    pl.kernel,
    out_type=jax.ShapeDtypeStruct((B,), jnp.float32),
    mesh=plsc.VectorSubcoreMesh(core_axis_name="c", subcore_axis_name="s"),
    compiler_params=pltpu.CompilerParams(needs_layout_passes=False),
    scratch_types=[
        pltpu.VMEM((T, BPW), jnp.int32),
        pltpu.VMEM((T, BPW), jnp.int32),
        pltpu.VMEM((ROWS, H), jnp.float32),
        pltpu.VMEM((T * H,), jnp.float32),
        pltpu.VMEM((T * H, L), jnp.float32),
        pltpu.VMEM((BPW,), jnp.float32),
        pltpu.SemaphoreType.DMA,
    ],
)(_sc_body)


def _tp_body(in_ref, out_ref):
    out_ref[:] = in_ref[:].T


_TPB = 2048
_tp_call = pl.pallas_call(
    _tp_body,
    grid=((VOCAB + _TPB - 1) // _TPB,),
    in_specs=[pl.BlockSpec((H, _TPB), lambda i: (0, i))],
    out_specs=pl.BlockSpec((_TPB, H), lambda i: (i, 0)),
    out_shape=jax.ShapeDtypeStruct((VOCAB, H), jnp.float32),
)


def _loss_body(pred_ref, lab_ref, b_ref, out_ref):
    p = pred_ref[:] + b_ref[0]
    lab = lab_ref[:]
    terms = (jnp.maximum(p, 0.0) - p * lab
             + jnp.log(1.0 + jnp.exp(-jnp.abs(p))))
    out_ref[0, 0] = jnp.sum(terms) * (1.0 / B)


_loss_call = pl.pallas_call(
    _loss_body,
    out_shape=jax.ShapeDtypeStruct((1, 1), jnp.float32),
    in_specs=[
        pl.BlockSpec(memory_space=pltpu.VMEM),
        pl.BlockSpec(memory_space=pltpu.VMEM),
        pl.BlockSpec(memory_space=pltpu.SMEM),
    ],
    out_specs=pl.BlockSpec(memory_space=pltpu.SMEM),
)


def kernel(seqs, targets, labels, table, W, b):
    seqs_flat = seqs.reshape(-1).astype(jnp.int32)
    tgt_flat = targets.reshape(-1).astype(jnp.int32)
    w_flat = W.reshape(-1)
    tableF = _tp_call(table.T)
    pred = _sc_gather(seqs_flat, tgt_flat, tableF, w_flat)
    loss = _loss_call(pred.reshape(B // 128, 128),
                      labels.reshape(B // 128, 128), b)
    return loss[0, 0]


# speedup vs baseline: 3.3022x; 1.3219x over previous
import functools

import jax
import jax.numpy as jnp
from jax import lax
from jax.experimental import pallas as pl
from jax.experimental.pallas import tpu as pltpu
from jax.experimental.pallas import tpu_sc as plsc

VOCAB = 1000000
H = 64
S = 200
B = 4096
T = 4
L = 16
NC = 2
NS = 16
NW = NC * NS
BPW = B // NW
NCHUNK = BPW // L
ROWS = T * BPW


def _sc_body(seqs_hbm, tgt_hbm, table_hbm, w_hbm, out_hbm,
             idx_v, tok_v, rows_v, w_v, wbc_v, pred_v, sem):
    wid = lax.axis_index("s") * NC + lax.axis_index("c")
    base = wid * BPW

    for t in range(T):
        pltpu.sync_copy(tgt_hbm.at[pl.ds(t * B + base, BPW)], idx_v.at[t])
    pltpu.sync_copy(w_hbm, w_v)
    for t in range(T):
        for j in range(NCHUNK):
            col = lax.iota(jnp.int32, L) + (base + j * L)
            sl = pl.ds(j * L, L)
            idx_v[t, sl] = idx_v[t, sl] * B + col

    cps = [pltpu.async_copy(seqs_hbm.at[idx_v.at[t]], tok_v.at[t], sem)
           for t in range(T)]
    for cp in cps:
        cp.wait()

    def wfill(j, carry):
        wbc_v[j, :] = plsc.load_gather(w_v, [jnp.full((L,), j, jnp.int32)])
        return carry
    lax.fori_loop(0, T * H, wfill, 0)

    def fetch(k, carry):
        t = k // NCHUNK
        tokc = tok_v[t, pl.ds((k % NCHUNK) * L, L)]
        for l in range(L):
            pltpu.async_copy(table_hbm.at[tokc[l]], rows_v.at[k * L + l], sem)
        return carry
    lax.fori_loop(0, T * NCHUNK, fetch, 0)
    pltpu.make_async_copy(table_hbm.at[pl.ds(0, ROWS)], rows_v, sem).wait()

    for t in range(T):
        iidx = [lax.iota(jnp.int32, L) + (t * BPW + c * L)
                for c in range(NCHUNK)]

        def hbody(h, accs, t=t, iidx=iidx):
            bw = wbc_v[t * H + h, :]
            colh = jnp.full((L,), h, jnp.int32)
            return tuple(
                accs[c] + plsc.load_gather(rows_v, [iidx[c], colh]) * bw
                for c in range(NCHUNK))

        accs = lax.fori_loop(
            0, H, hbody, tuple(jnp.zeros((L,), jnp.float32)
                               for _ in range(NCHUNK)))
        for c in range(NCHUNK):
            sl = pl.ds(c * L, L)
            if t == 0:
                pred_v[sl] = accs[c]
            else:
                pred_v[sl] = pred_v[sl] + accs[c]

    pltpu.sync_copy(pred_v, out_hbm.at[pl.ds(base, BPW)])


_sc_gather = functools.partial(
    pl.kernel,
    out_type=jax.ShapeDtypeStruct((B,), jnp.float32),
    mesh=plsc.VectorSubcoreMesh(core_axis_name="c", subcore_axis_name="s"),
    compiler_params=pltpu.CompilerParams(needs_layout_passes=False),
    scratch_types=[
        pltpu.VMEM((T, BPW), jnp.int32),
        pltpu.VMEM((T, BPW), jnp.int32),
        pltpu.VMEM((ROWS, H), jnp.float32),
        pltpu.VMEM((T * H,), jnp.float32),
        pltpu.VMEM((T * H, L), jnp.float32),
        pltpu.VMEM((BPW,), jnp.float32),
        pltpu.SemaphoreType.DMA,
    ],
)(_sc_body)


def _loss_body(pred_ref, lab_ref, b_ref, out_ref):
    p = pred_ref[:] + b_ref[0]
    lab = lab_ref[:]
    terms = (jnp.maximum(p, 0.0) - p * lab
             + jnp.log(1.0 + jnp.exp(-jnp.abs(p))))
    out_ref[0, 0] = jnp.sum(terms) * (1.0 / B)


_loss_call = pl.pallas_call(
    _loss_body,
    out_shape=jax.ShapeDtypeStruct((1, 1), jnp.float32),
    in_specs=[
        pl.BlockSpec(memory_space=pltpu.VMEM),
        pl.BlockSpec(memory_space=pltpu.VMEM),
        pl.BlockSpec(memory_space=pltpu.SMEM),
    ],
    out_specs=pl.BlockSpec(memory_space=pltpu.SMEM),
)


def kernel(seqs, targets, labels, table, W, b):
    seqs_flat = seqs.reshape(-1).astype(jnp.int32)
    tgt_flat = targets.reshape(-1).astype(jnp.int32)
    w_flat = W.reshape(-1)
    pred = _sc_gather(seqs_flat, tgt_flat, table, w_flat)
    loss = _loss_call(pred.reshape(B // 128, 128),
                      labels.reshape(B // 128, 128), b)
    return loss[0, 0]
